# baseline (speedup 1.0000x reference)
import functools

import jax
import jax.numpy as jnp
from jax import lax
from jax.experimental import pallas as pl
from jax.experimental.pallas import tpu as pltpu


def _round_up(v, m):
    return ((v + m - 1) // m) * m


def _gate_block(x, wa, ba, ws, bs, hw_real):
    s = jnp.sum(x, axis=1)
    ss = jnp.sum(x * x, axis=1)
    hw = jnp.float32(hw_real)
    denom = jnp.float32(max(hw_real - 1, 1))
    mean = s / hw
    var = jnp.maximum((ss - hw * mean * mean) / denom, 0.0)
    si = jnp.sqrt(var)
    dn = (((1,), (1,)), ((), ()))
    a = jnp.maximum(
        lax.dot_general(mean, wa, dn, preferred_element_type=jnp.float32) + ba,
        0.0)
    b = jnp.maximum(
        lax.dot_general(si, ws, dn, preferred_element_type=jnp.float32) + bs,
        0.0)
    gate = jax.nn.sigmoid(a + b)
    return x * gate[:, None, :]


def _manual_kernel(x_hbm, wa_ref, ba_ref, ws_ref, bs_ref, o_hbm,
                   xbuf, obuf, insem, outsem, *, hw_real, cb, depth, nsteps):
    wa = wa_ref[...]
    ba = ba_ref[...]
    ws = ws_ref[...]
    bs = bs_ref[...]

    def in_copy(step, slot):
        return pltpu.make_async_copy(
            x_hbm.at[pl.ds(step * cb, cb)], xbuf.at[slot], insem.at[slot])

    def out_copy(step, slot):
        return pltpu.make_async_copy(
            obuf.at[slot], o_hbm.at[pl.ds(step * cb, cb)], outsem.at[slot])

    for s in range(min(depth - 1, nsteps)):
        in_copy(s, s).start()

    def body(k, carry):
        slot = lax.rem(k, depth)
        ahead = k + depth - 1
        @pl.when(ahead < nsteps)
        def _():
            in_copy(ahead, lax.rem(ahead, depth)).start()

        in_copy(k, slot).wait()
        @pl.when(k >= depth)
        def _():
            out_copy(k - depth, slot).wait()

        obuf[slot] = _gate_block(xbuf[slot], wa, ba, ws, bs, hw_real)
        out_copy(k, slot).start()
        return carry

    lax.fori_loop(0, nsteps, body, 0, unroll=False)

    for s in range(min(depth, nsteps)):
        step = nsteps - min(depth, nsteps) + s
        out_copy(step, step % depth).wait()


def kernel(x, w_avg, b_avg, w_si, b_si):
    B, C, H, W = x.shape
    HW = H * W
    hw_pad = _round_up(HW, 8)
    c_pad = _round_up(C, 128)

    xt = jnp.transpose(x, (0, 2, 3, 1)).reshape(B, HW, C)
    if hw_pad != HW or c_pad != C:
        xt = jnp.pad(xt, ((0, 0), (0, hw_pad - HW), (0, c_pad - C)))

    wa = jnp.asarray(w_avg)
    ws = jnp.asarray(w_si)
    ba = jnp.asarray(b_avg).reshape(1, C)
    bs = jnp.asarray(b_si).reshape(1, C)
    if c_pad != C:
        wa = jnp.pad(wa, ((0, c_pad - C), (0, c_pad - C)))
        ws = jnp.pad(ws, ((0, c_pad - C), (0, c_pad - C)))
        ba = jnp.pad(ba, ((0, 0), (0, c_pad - C)))
        bs = jnp.pad(bs, ((0, 0), (0, c_pad - C)))

    itemsize = jnp.dtype(x.dtype).itemsize
    slab = c_pad * hw_pad * itemsize
    cb = 1
    while cb < B and B % (cb * 2) == 0 and (cb * 2) * slab <= (4 << 20):
        cb *= 2
    nsteps = B // cb
    depth = min(6, nsteps)

    out = pl.pallas_call(
        functools.partial(_manual_kernel, hw_real=HW, cb=cb, depth=depth,
                          nsteps=nsteps),
        out_shape=jax.ShapeDtypeStruct((B, hw_pad, c_pad), x.dtype),
        grid=(1,),
        in_specs=[
            pl.BlockSpec(memory_space=pltpu.MemorySpace.HBM),
            pl.BlockSpec((c_pad, c_pad), lambda i: (0, 0)),
            pl.BlockSpec((1, c_pad), lambda i: (0, 0)),
            pl.BlockSpec((c_pad, c_pad), lambda i: (0, 0)),
            pl.BlockSpec((1, c_pad), lambda i: (0, 0)),
        ],
        out_specs=pl.BlockSpec(memory_space=pltpu.MemorySpace.HBM),
        scratch_shapes=[
            pltpu.VMEM((depth, cb, hw_pad, c_pad), x.dtype),
            pltpu.VMEM((depth, cb, hw_pad, c_pad), x.dtype),
            pltpu.SemaphoreType.DMA((depth,)),
            pltpu.SemaphoreType.DMA((depth,)),
        ],
        compiler_params=pltpu.CompilerParams(
            dimension_semantics=("arbitrary",), vmem_limit_bytes=64 << 20),
    )(xt, wa, ba, ws, bs)

    if hw_pad != HW or c_pad != C:
        out = out[:, :HW, :C]
    return out.reshape(B, H, W, C).transpose(0, 3, 1, 2)


# speedup vs baseline: 1.0086x; 1.0086x over previous
import functools

import jax
import jax.numpy as jnp
from jax import lax
from jax.experimental import pallas as pl
from jax.experimental.pallas import tpu as pltpu


def _round_up(v, m):
    return ((v + m - 1) // m) * m


def _gate_block(x, wa, ba, ws, bs, hw_real):
    s = jnp.sum(x, axis=1)
    ss = jnp.sum(x * x, axis=1)
    hw = jnp.float32(hw_real)
    denom = jnp.float32(max(hw_real - 1, 1))
    mean = s / hw
    var = jnp.maximum((ss - hw * mean * mean) / denom, 0.0)
    si = jnp.sqrt(var)
    dn = (((1,), (1,)), ((), ()))
    a = jnp.maximum(
        lax.dot_general(mean, wa, dn, preferred_element_type=jnp.float32) + ba,
        0.0)
    b = jnp.maximum(
        lax.dot_general(si, ws, dn, preferred_element_type=jnp.float32) + bs,
        0.0)
    gate = jax.nn.sigmoid(a + b)
    return x * gate[:, None, :]


def _manual_kernel(x_hbm, wa_ref, ba_ref, ws_ref, bs_ref, o_hbm,
                   xbuf, obuf, insem, outsem, *, hw_real, cb, depth, nsteps):
    wa = wa_ref[...]
    ba = ba_ref[...]
    ws = ws_ref[...]
    bs = bs_ref[...]

    def in_copy(step, slot):
        return pltpu.make_async_copy(
            x_hbm.at[pl.ds(step * cb, cb)], xbuf.at[slot], insem.at[slot])

    def out_copy(step, slot):
        return pltpu.make_async_copy(
            obuf.at[slot], o_hbm.at[pl.ds(step * cb, cb)], outsem.at[slot])

    for s in range(min(depth - 1, nsteps)):
        in_copy(s, s).start()

    def body(k, carry):
        slot = lax.rem(k, depth)
        ahead = k + depth - 1
        @pl.when(ahead < nsteps)
        def _():
            in_copy(ahead, lax.rem(ahead, depth)).start()

        in_copy(k, slot).wait()
        @pl.when(k >= depth)
        def _():
            out_copy(k - depth, slot).wait()

        obuf[slot] = _gate_block(xbuf[slot], wa, ba, ws, bs, hw_real)
        out_copy(k, slot).start()
        return carry

    lax.fori_loop(0, nsteps, body, 0, unroll=False)

    for s in range(min(depth, nsteps)):
        step = nsteps - min(depth, nsteps) + s
        out_copy(step, step % depth).wait()


def kernel(x, w_avg, b_avg, w_si, b_si):
    B, C, H, W = x.shape
    HW = H * W
    hw_pad = _round_up(HW, 8)
    c_pad = _round_up(C, 128)

    xt = jnp.transpose(x, (0, 2, 3, 1)).reshape(B, HW, C)
    if hw_pad != HW or c_pad != C:
        xt = jnp.pad(xt, ((0, 0), (0, hw_pad - HW), (0, c_pad - C)))

    wa = jnp.asarray(w_avg)
    ws = jnp.asarray(w_si)
    ba = jnp.asarray(b_avg).reshape(1, C)
    bs = jnp.asarray(b_si).reshape(1, C)
    if c_pad != C:
        wa = jnp.pad(wa, ((0, c_pad - C), (0, c_pad - C)))
        ws = jnp.pad(ws, ((0, c_pad - C), (0, c_pad - C)))
        ba = jnp.pad(ba, ((0, 0), (0, c_pad - C)))
        bs = jnp.pad(bs, ((0, 0), (0, c_pad - C)))

    itemsize = jnp.dtype(x.dtype).itemsize
    slab = c_pad * hw_pad * itemsize
    cb = 1
    while cb < B and B % (cb * 2) == 0 and (cb * 2) * slab <= (4 << 20):
        cb *= 2
    nsteps = B // cb
    depth = min(4, nsteps)

    out = pl.pallas_call(
        functools.partial(_manual_kernel, hw_real=HW, cb=cb, depth=depth,
                          nsteps=nsteps),
        out_shape=jax.ShapeDtypeStruct((B, hw_pad, c_pad), x.dtype),
        grid=(1,),
        in_specs=[
            pl.BlockSpec(memory_space=pltpu.MemorySpace.HBM),
            pl.BlockSpec((c_pad, c_pad), lambda i: (0, 0)),
            pl.BlockSpec((1, c_pad), lambda i: (0, 0)),
            pl.BlockSpec((c_pad, c_pad), lambda i: (0, 0)),
            pl.BlockSpec((1, c_pad), lambda i: (0, 0)),
        ],
        out_specs=pl.BlockSpec(memory_space=pltpu.MemorySpace.HBM),
        scratch_shapes=[
            pltpu.VMEM((depth, cb, hw_pad, c_pad), x.dtype),
            pltpu.VMEM((depth, cb, hw_pad, c_pad), x.dtype),
            pltpu.SemaphoreType.DMA((depth,)),
            pltpu.SemaphoreType.DMA((depth,)),
        ],
        compiler_params=pltpu.CompilerParams(
            dimension_semantics=("arbitrary",), vmem_limit_bytes=64 << 20),
    )(xt, wa, ba, ws, bs)

    if hw_pad != HW or c_pad != C:
        out = out[:, :HW, :C]
    return out.reshape(B, H, W, C).transpose(0, 3, 1, 2)
